# Initial kernel scaffold; baseline (speedup 1.0000x reference)
import jax
import jax.numpy as jnp
from jax.experimental import pallas as pl


def kernel(x, estimated_p):
    raise NotImplementedError("write your pallas kernel here")


# speedup vs baseline: 238.8407x; 238.8407x over previous
import functools

import jax
import jax.numpy as jnp
from jax import lax
from jax.experimental import pallas as pl
from jax.experimental.pallas import tpu as pltpu
from jax.experimental.pallas import tpu_sc as plsc

MOM = 0.99
N, C, H, W = 4, 96, 224, 224
HW = H * W
PER_CH = N * HW
NC, NS, L = 2, 16, 16
NW = NC * NS
CPT = C // NW
CH = HW // 4
NCHUNK = CPT * N * 4
GROUPS = CH // L
UNROLL = 8
BINS_T = CPT * 256
MAGIC = 8388608.0


def _flat_off(i, c0):
    ch_l = i >> 4
    r = i & 15
    n = r >> 2
    ck = r & 3
    row = n * C + c0 + ch_l
    return row * HW + ck * CH, ch_l


def _sc_body(x_hbm, ep_hbm, xste_hbm, px_hbm,
             in0, in1, out0, out1, hist16, table, ep_v,
             si0, si1, so0, so1):
    wid = lax.axis_index("s") * NC + lax.axis_index("c")
    c0 = wid * CPT

    lane = lax.iota(jnp.int32, 16)
    lane768 = lane * BINS_T
    ones = jnp.full((16,), 1.0, jnp.float32)
    zeros = jnp.zeros((16,), jnp.float32)

    def zbody(g, _):
        hist16[pl.ds(g * 16, 16)] = zeros
    lax.fori_loop(0, L * BINS_T // 16, zbody, None)

    def start_in(src_hbm, i, buf, sem):
        off, _ = _flat_off(i, c0)
        pltpu.async_copy(src_hbm.at[pl.ds(off, CH)], buf, sem)

    def wait_dma(src_hbm, buf, sem):
        pltpu.make_async_copy(src_hbm.at[pl.ds(0, CH)], buf, sem).wait()

    def start_out(dst_hbm, i, buf, sem):
        off, _ = _flat_off(i, c0)
        pltpu.async_copy(buf, dst_hbm.at[pl.ds(off, CH)], sem)

    def wait_out(dst_hbm, buf, sem):
        pltpu.make_async_copy(buf, dst_hbm.at[pl.ds(0, CH)], sem).wait()

    def p1_compute(i, ibuf, obuf):
        _, ch_l = _flat_off(i, c0)
        base = lane768 + ch_l * 256

        def body(u, _):
            for k in range(UNROLL):
                g = u * UNROLL + k
                v = ibuf[pl.ds(g * 16, 16)]
                v = jnp.minimum(v, 255.0)
                rv = (v + MAGIC) - MAGIC
                obuf[pl.ds(g * 16, 16)] = rv
                b = rv.astype(jnp.int32)
                plsc.addupdate_scatter(hist16, [b + base], ones)
            return None
        lax.fori_loop(0, GROUPS // UNROLL, body, None)

    start_in(x_hbm, 0, in0, si0)
    start_in(x_hbm, 1, in1, si1)

    def p1_iter(j, _):
        ia = 2 * j
        ib = ia + 1
        wait_dma(x_hbm, in0, si0)
        pl.when(j > 0)(lambda: wait_out(xste_hbm, out0, so0))
        p1_compute(ia, in0, out0)
        pl.when(ia + 2 < NCHUNK)(lambda: start_in(x_hbm, ia + 2, in0, si0))
        start_out(xste_hbm, ia, out0, so0)

        wait_dma(x_hbm, in1, si1)
        pl.when(j > 0)(lambda: wait_out(xste_hbm, out1, so1))
        p1_compute(ib, in1, out1)
        pl.when(ib + 2 < NCHUNK)(lambda: start_in(x_hbm, ib + 2, in1, si1))
        start_out(xste_hbm, ib, out1, so1)
        return None

    lax.fori_loop(0, NCHUNK // 2, p1_iter, None)
    wait_out(xste_hbm, out0, so0)
    wait_out(xste_hbm, out1, so1)

    pltpu.sync_copy(ep_hbm.at[pl.ds(c0 * 256, BINS_T)], ep_v)

    def ema_body(g, _):
        acc = zeros
        for l in range(L):
            acc = acc + hist16[pl.ds(l * BINS_T + g * 16, 16)]
        e = ep_v[pl.ds(g * 16, 16)]
        table[pl.ds(g * 16, 16)] = e * MOM + acc * ((1.0 - MOM) / PER_CH)
        return None
    lax.fori_loop(0, BINS_T // 16, ema_body, None)

    def p2_compute(i, ibuf, obuf):
        _, ch_l = _flat_off(i, c0)
        toff = ch_l * 256

        def body(u, _):
            for k in range(UNROLL):
                g = u * UNROLL + k
                rv = ibuf[pl.ds(g * 16, 16)]
                b = rv.astype(jnp.int32) + toff
                obuf[pl.ds(g * 16, 16)] = plsc.load_gather(table, [b])
            return None
        lax.fori_loop(0, GROUPS // UNROLL, body, None)

    start_in(xste_hbm, 0, in0, si0)
    start_in(xste_hbm, 1, in1, si1)

    def p2_iter(j, _):
        ia = 2 * j
        ib = ia + 1
        wait_dma(xste_hbm, in0, si0)
        pl.when(j > 0)(lambda: wait_out(px_hbm, out0, so0))
        p2_compute(ia, in0, out0)
        pl.when(ia + 2 < NCHUNK)(lambda: start_in(xste_hbm, ia + 2, in0, si0))
        start_out(px_hbm, ia, out0, so0)

        wait_dma(xste_hbm, in1, si1)
        pl.when(j > 0)(lambda: wait_out(px_hbm, out1, so1))
        p2_compute(ib, in1, out1)
        pl.when(ib + 2 < NCHUNK)(lambda: start_in(xste_hbm, ib + 2, in1, si1))
        start_out(px_hbm, ib, out1, so1)
        return None

    lax.fori_loop(0, NCHUNK // 2, p2_iter, None)
    wait_out(px_hbm, out0, so0)
    wait_out(px_hbm, out1, so1)


@jax.jit
def kernel(x, estimated_p):
    total = N * C * HW
    xf = x.reshape(total)
    epf = estimated_p.reshape(C * 256)

    mesh = plsc.VectorSubcoreMesh(core_axis_name="c", subcore_axis_name="s")
    run = functools.partial(
        pl.kernel,
        out_type=[
            jax.ShapeDtypeStruct((total,), jnp.float32),
            jax.ShapeDtypeStruct((total,), jnp.float32),
        ],
        mesh=mesh,
        compiler_params=pltpu.CompilerParams(needs_layout_passes=False),
        scratch_types=[
            pltpu.VMEM((CH,), jnp.float32),
            pltpu.VMEM((CH,), jnp.float32),
            pltpu.VMEM((CH,), jnp.float32),
            pltpu.VMEM((CH,), jnp.float32),
            pltpu.VMEM((L * BINS_T,), jnp.float32),
            pltpu.VMEM((BINS_T,), jnp.float32),
            pltpu.VMEM((BINS_T,), jnp.float32),
            pltpu.SemaphoreType.DMA,
            pltpu.SemaphoreType.DMA,
            pltpu.SemaphoreType.DMA,
            pltpu.SemaphoreType.DMA,
        ],
    )(_sc_body)
    xste, px = run(xf, epf)
    shape = (N, C, H, W)
    return xste.reshape(shape), px.reshape(shape)
